# baseline (speedup 1.0000x reference)
import math

import jax
import jax.numpy as jnp
from jax.experimental import pallas as pl
from jax.experimental.pallas import tpu as pltpu

_B, _S, _D, _H, _KVH, _HD = 2, 1024, 4096, 32, 8, 128
_NREP = _H // _KVH
_M = _B * _S
_NQ = _H * _HD
_NKV = _KVH * _HD
_NTOT = _NQ + 2 * _NKV
_TN = 512
_TQ = 512
_TO = 1024


def _qkv_body(x_ref, w_ref, cos_ref, sin_ref, o_ref):
    n_half = _NTOT // _TN // 2
    t = pl.program_id(0) * n_half + pl.program_id(1)
    acc = jnp.dot(x_ref[...], w_ref[...], preferred_element_type=jnp.float32)

    @pl.when(t < (_NQ + _NKV) // _TN)
    def _():
        parts = []
        for g in range(_TN // _HD):
            lo = acc[:, g * _HD : g * _HD + _HD // 2]
            hi = acc[:, g * _HD + _HD // 2 : (g + 1) * _HD]
            parts.append(hi)
            parts.append(lo)
        rot = jnp.concatenate(parts, axis=1)
        cos_t = jnp.tile(cos_ref[...], (_B, _TN // _HD))
        sin_t = jnp.tile(sin_ref[...], (_B, _TN // _HD))
        o_ref[...] = (acc * cos_t + rot * sin_t).astype(o_ref.dtype)

    @pl.when(t >= (_NQ + _NKV) // _TN)
    def _():
        o_ref[...] = acc.astype(o_ref.dtype)


def _attn_body(q_ref, k_ref, v_ref, o_ref):
    qt = pl.program_id(2)
    q = q_ref[...]
    k = k_ref[...]
    s = jax.lax.dot_general(
        q, k, (((1,), (1,)), ((), ())),
        preferred_element_type=jnp.float32)
    qpos = qt * _TQ + jax.lax.broadcasted_iota(jnp.int32, (_TQ, _S), 0)
    kpos = jax.lax.broadcasted_iota(jnp.int32, (_TQ, _S), 1)
    s = jnp.where(kpos > qpos, -1e9, s)
    m = jnp.max(s, axis=-1, keepdims=True)
    p = jnp.exp(s - m)
    l = jnp.sum(p, axis=-1, keepdims=True)
    o = jnp.dot(p.astype(jnp.bfloat16), v_ref[...],
                preferred_element_type=jnp.float32)
    o_ref[...] = (o / l).astype(o_ref.dtype)


def _proj_body(x_ref, w_ref, o_ref):
    o_ref[...] = jnp.dot(x_ref[...], w_ref[...],
                         preferred_element_type=jnp.float32)


def _rope_perm(w):
    d, n = w.shape
    return w.reshape(d, n // _HD, _HD // 2, 2).transpose(0, 1, 3, 2).reshape(d, n)


def kernel(x, start_pos, cos, sin, mask, wq, wk, wv, wo):
    del start_pos, mask
    scale = 1.0 / math.sqrt(_HD)
    wqkv = jnp.concatenate(
        [_rope_perm(wq) * scale, _rope_perm(wk), wv], axis=1
    ).astype(jnp.bfloat16)
    xb = x.reshape(_M, _D).astype(jnp.bfloat16)
    cos_a = jnp.concatenate([cos, cos], axis=1)
    sin_a = jnp.concatenate([-sin, sin], axis=1)

    qkv = pl.pallas_call(
        _qkv_body,
        grid=(2, _NTOT // _TN // 2),
        in_specs=[
            pl.BlockSpec((_M, _D), lambda c, n: (0, 0)),
            pl.BlockSpec((_D, _TN), lambda c, n: (0, c * (_NTOT // _TN // 2) + n)),
            pl.BlockSpec((_S, _HD), lambda c, n: (0, 0)),
            pl.BlockSpec((_S, _HD), lambda c, n: (0, 0)),
        ],
        out_specs=pl.BlockSpec((_M, _TN), lambda c, n: (0, c * (_NTOT // _TN // 2) + n)),
        out_shape=jax.ShapeDtypeStruct((_M, _NTOT), jnp.bfloat16),
        compiler_params=pltpu.CompilerParams(
            dimension_semantics=(pltpu.PARALLEL, pltpu.ARBITRARY),
        ),
    )(xb, wqkv, cos_a, sin_a)

    attn = pl.pallas_call(
        _attn_body,
        grid=(2, _H, _S // _TQ),
        in_specs=[
            pl.BlockSpec((_TQ, _HD), lambda b, h, qt: (b * (_S // _TQ) + qt, h)),
            pl.BlockSpec((_S, _HD), lambda b, h, qt: (b, _H + h // _NREP)),
            pl.BlockSpec((_S, _HD), lambda b, h, qt: (b, _H + _KVH + h // _NREP)),
        ],
        out_specs=pl.BlockSpec((_TQ, _HD), lambda b, h, qt: (b * (_S // _TQ) + qt, h)),
        out_shape=jax.ShapeDtypeStruct((_M, _NQ), jnp.bfloat16),
        compiler_params=pltpu.CompilerParams(
            dimension_semantics=(pltpu.PARALLEL, pltpu.ARBITRARY, pltpu.ARBITRARY),
        ),
    )(qkv, qkv, qkv)

    wo_b = wo.astype(jnp.bfloat16)
    out = pl.pallas_call(
        _proj_body,
        grid=(2, _D // _TO),
        in_specs=[
            pl.BlockSpec((_M // 2, _NQ), lambda m, j: (m, 0)),
            pl.BlockSpec((_NQ, _TO), lambda m, j: (0, j)),
        ],
        out_specs=pl.BlockSpec((_M // 2, _TO), lambda m, j: (m, j)),
        out_shape=jax.ShapeDtypeStruct((_M, _D), jnp.float32),
        compiler_params=pltpu.CompilerParams(
            dimension_semantics=(pltpu.PARALLEL, pltpu.ARBITRARY),
        ),
    )(attn, wo_b)

    return out.reshape(_B, _S, _D)


# speedup vs baseline: 1.6056x; 1.6056x over previous
import math

import jax
import jax.numpy as jnp
import numpy as np
from jax.experimental import pallas as pl
from jax.experimental.pallas import tpu as pltpu

_B, _S, _D, _H, _KVH, _HD = 2, 1024, 4096, 32, 8, 128
_NREP = _H // _KVH
_M = _B * _S
_NQ = _H * _HD
_NKV = _KVH * _HD
_NTOT = _NQ + 2 * _NKV
_TN = 256
_NT_Q = _NQ // _TN
_NT_K = _NKV // _TN
_NT_HALF = _NTOT // _TN // 2
_TQ = 512
_TO = 512

_PSWAP = np.zeros((_TN, _TN), dtype=np.float32)
for _g in range(_TN // _HD):
    for _j in range(_HD // 2):
        _PSWAP[_g * _HD + 2 * _j, _g * _HD + 2 * _j + 1] = 1.0
        _PSWAP[_g * _HD + 2 * _j + 1, _g * _HD + 2 * _j] = 1.0


def _qkv_body(x_ref, wq_ref, wk_ref, wv_ref, p_ref, cos_ref, sin_ref, o_ref):
    t = pl.program_id(0) * _NT_HALF + pl.program_id(1)
    scale = 1.0 / math.sqrt(_HD)

    def rope(acc):
        rot = jnp.dot(acc.astype(jnp.bfloat16), p_ref[...],
                      preferred_element_type=jnp.float32)
        cos_t = jnp.tile(cos_ref[...], (_B, _TN // _HD))
        sin_t = jnp.tile(sin_ref[...], (_B, _TN // _HD))
        return acc * cos_t + rot * sin_t

    @pl.when(t < _NT_Q)
    def _():
        acc = jnp.dot(x_ref[...], wq_ref[...].astype(jnp.bfloat16),
                      preferred_element_type=jnp.float32)
        o_ref[...] = (rope(acc) * scale).astype(o_ref.dtype)

    @pl.when(jnp.logical_and(t >= _NT_Q, t < _NT_Q + _NT_K))
    def _():
        acc = jnp.dot(x_ref[...], wk_ref[...].astype(jnp.bfloat16),
                      preferred_element_type=jnp.float32)
        o_ref[...] = rope(acc).astype(o_ref.dtype)

    @pl.when(t >= _NT_Q + _NT_K)
    def _():
        acc = jnp.dot(x_ref[...], wv_ref[...].astype(jnp.bfloat16),
                      preferred_element_type=jnp.float32)
        o_ref[...] = acc.astype(o_ref.dtype)


def _attn_body(q_ref, k_ref, v_ref, o_ref):
    qt = pl.program_id(2)
    q = q_ref[...]
    k = k_ref[...]
    s = jax.lax.dot_general(
        q, k, (((1,), (1,)), ((), ())),
        preferred_element_type=jnp.float32)
    qpos = qt * _TQ + jax.lax.broadcasted_iota(jnp.int32, (_TQ, _S), 0)
    kpos = jax.lax.broadcasted_iota(jnp.int32, (_TQ, _S), 1)
    s = jnp.where(kpos > qpos, -1e9, s)
    m = jnp.max(s, axis=-1, keepdims=True)
    p = jnp.exp(s - m)
    l = jnp.sum(p, axis=-1, keepdims=True)
    o = jnp.dot(p.astype(jnp.bfloat16), v_ref[...],
                preferred_element_type=jnp.float32)
    o_ref[...] = (o / l).astype(o_ref.dtype)


def _proj_body(x_ref, w_ref, o_ref):
    o_ref[...] = jnp.dot(x_ref[...], w_ref[...].astype(jnp.bfloat16),
                         preferred_element_type=jnp.float32)


def kernel(x, start_pos, cos, sin, mask, wq, wk, wv, wo):
    del start_pos, mask
    xb = x.reshape(_M, _D).astype(jnp.bfloat16)
    cos_i = jnp.stack([cos, cos], axis=-1).reshape(_S, _HD)
    sin_i = jnp.stack([-sin, sin], axis=-1).reshape(_S, _HD)
    pswap = jnp.asarray(_PSWAP, dtype=jnp.bfloat16)

    qkv = pl.pallas_call(
        _qkv_body,
        grid=(2, _NT_HALF),
        in_specs=[
            pl.BlockSpec((_M, _D), lambda c, n: (0, 0)),
            pl.BlockSpec((_D, _TN),
                         lambda c, n: (0, jnp.minimum(c * _NT_HALF + n, _NT_Q - 1))),
            pl.BlockSpec((_D, _TN),
                         lambda c, n: (0, jnp.clip(c * _NT_HALF + n - _NT_Q, 0, _NT_K - 1))),
            pl.BlockSpec((_D, _TN),
                         lambda c, n: (0, jnp.clip(c * _NT_HALF + n - _NT_Q - _NT_K, 0, _NT_K - 1))),
            pl.BlockSpec((_TN, _TN), lambda c, n: (0, 0)),
            pl.BlockSpec((_S, _HD), lambda c, n: (0, 0)),
            pl.BlockSpec((_S, _HD), lambda c, n: (0, 0)),
        ],
        out_specs=pl.BlockSpec((_M, _TN), lambda c, n: (0, c * _NT_HALF + n)),
        out_shape=jax.ShapeDtypeStruct((_M, _NTOT), jnp.bfloat16),
        compiler_params=pltpu.CompilerParams(
            dimension_semantics=(pltpu.PARALLEL, pltpu.ARBITRARY),
        ),
    )(xb, wq, wk, wv, pswap, cos_i, sin_i)

    attn = pl.pallas_call(
        _attn_body,
        grid=(2, _H, _S // _TQ),
        in_specs=[
            pl.BlockSpec((_TQ, _HD), lambda b, h, qt: (b * (_S // _TQ) + qt, h)),
            pl.BlockSpec((_S, _HD), lambda b, h, qt: (b, _H + h // _NREP)),
            pl.BlockSpec((_S, _HD), lambda b, h, qt: (b, _H + _KVH + h // _NREP)),
        ],
        out_specs=pl.BlockSpec((_TQ, _HD), lambda b, h, qt: (b * (_S // _TQ) + qt, h)),
        out_shape=jax.ShapeDtypeStruct((_M, _NQ), jnp.bfloat16),
        compiler_params=pltpu.CompilerParams(
            dimension_semantics=(pltpu.PARALLEL, pltpu.ARBITRARY, pltpu.ARBITRARY),
        ),
    )(qkv, qkv, qkv)

    out = pl.pallas_call(
        _proj_body,
        grid=(_D // _TO,),
        in_specs=[
            pl.BlockSpec((_M, _NQ), lambda j: (0, 0)),
            pl.BlockSpec((_NQ, _TO), lambda j: (0, j)),
        ],
        out_specs=pl.BlockSpec((_M, _TO), lambda j: (0, j)),
        out_shape=jax.ShapeDtypeStruct((_M, _D), jnp.float32),
        compiler_params=pltpu.CompilerParams(
            dimension_semantics=(pltpu.ARBITRARY,),
        ),
    )(attn, wo)

    return out.reshape(_B, _S, _D)


# speedup vs baseline: 1.9298x; 1.2019x over previous
import math

import jax
import jax.numpy as jnp
import numpy as np
from jax.experimental import pallas as pl
from jax.experimental.pallas import tpu as pltpu

_B, _S, _D, _H, _KVH, _HD = 2, 1024, 4096, 32, 8, 128
_NREP = _H // _KVH
_M = _B * _S
_NQ = _H * _HD
_NKV = _KVH * _HD
_NTOT = _NQ + 2 * _NKV
_TN = 256
_NT_Q = _NQ // _TN
_NT_K = _NKV // _TN
_NT = _NTOT // _TN
_TO = 512

_PSWAP = np.zeros((_TN, _TN), dtype=np.float32)
for _g in range(_TN // _HD):
    for _j in range(_HD // 2):
        _PSWAP[_g * _HD + 2 * _j, _g * _HD + 2 * _j + 1] = 1.0
        _PSWAP[_g * _HD + 2 * _j + 1, _g * _HD + 2 * _j] = 1.0


def _qkv_body(x_ref, wq_ref, wk_ref, wv_ref, p_ref, cos_ref, sin_ref, o_ref):
    t = pl.program_id(1)
    scale = 1.0 / math.sqrt(_HD)

    def rope(acc):
        rot = jnp.dot(acc.astype(jnp.bfloat16), p_ref[...],
                      preferred_element_type=jnp.float32)
        cos_t = jnp.tile(cos_ref[...], (1, _TN // _HD))
        sin_t = jnp.tile(sin_ref[...], (1, _TN // _HD))
        return acc * cos_t + rot * sin_t

    @pl.when(t < _NT_Q)
    def _():
        acc = jnp.dot(x_ref[...], wq_ref[...].astype(jnp.bfloat16),
                      preferred_element_type=jnp.float32)
        o_ref[...] = (rope(acc) * scale).astype(o_ref.dtype)

    @pl.when(jnp.logical_and(t >= _NT_Q, t < _NT_Q + _NT_K))
    def _():
        acc = jnp.dot(x_ref[...], wk_ref[...].astype(jnp.bfloat16),
                      preferred_element_type=jnp.float32)
        o_ref[...] = rope(acc).astype(o_ref.dtype)

    @pl.when(t >= _NT_Q + _NT_K)
    def _():
        acc = jnp.dot(x_ref[...], wv_ref[...].astype(jnp.bfloat16),
                      preferred_element_type=jnp.float32)
        o_ref[...] = acc.astype(o_ref.dtype)


def _attn_body(q_ref, k_ref, v_ref, o_ref):
    k = k_ref[...]
    v = v_ref[...]
    half = _S // 2

    def softmax_pv(s, kv_len, q0):
        qpos = q0 + jax.lax.broadcasted_iota(jnp.int32, s.shape, 0)
        kpos = jax.lax.broadcasted_iota(jnp.int32, s.shape, 1)
        s = jnp.where(kpos > qpos, -1e9, s)
        m = jnp.max(s, axis=-1, keepdims=True)
        p = jnp.exp(s - m)
        l = jnp.sum(p, axis=-1, keepdims=True)
        o = jnp.dot(p.astype(jnp.bfloat16), v[:kv_len],
                    preferred_element_type=jnp.float32)
        return o / l

    q_top = q_ref[:half, :]
    s_top = jax.lax.dot_general(q_top, k[:half], (((1,), (1,)), ((), ())),
                                preferred_element_type=jnp.float32)
    o_ref[:half, :] = softmax_pv(s_top, half, 0).astype(o_ref.dtype)

    q_bot = q_ref[half:, :]
    s_bot = jax.lax.dot_general(q_bot, k, (((1,), (1,)), ((), ())),
                                preferred_element_type=jnp.float32)
    o_ref[half:, :] = softmax_pv(s_bot, _S, half).astype(o_ref.dtype)


def _proj_body(x_ref, w_ref, o_ref):
    o_ref[...] = jnp.dot(x_ref[...], w_ref[...].astype(jnp.bfloat16),
                         preferred_element_type=jnp.float32)


def kernel(x, start_pos, cos, sin, mask, wq, wk, wv, wo):
    del start_pos, mask
    xb = x.reshape(_M, _D).astype(jnp.bfloat16)
    cos_i = jnp.stack([cos, cos], axis=-1).reshape(_S, _HD)
    sin_i = jnp.stack([-sin, sin], axis=-1).reshape(_S, _HD)
    pswap = jnp.asarray(_PSWAP, dtype=jnp.bfloat16)

    qkv = pl.pallas_call(
        _qkv_body,
        grid=(2, _NT),
        in_specs=[
            pl.BlockSpec((_M // 2, _D), lambda m, n: (m, 0)),
            pl.BlockSpec((_D, _TN), lambda m, n: (0, jnp.minimum(n, _NT_Q - 1))),
            pl.BlockSpec((_D, _TN),
                         lambda m, n: (0, jnp.clip(n - _NT_Q, 0, _NT_K - 1))),
            pl.BlockSpec((_D, _TN),
                         lambda m, n: (0, jnp.clip(n - _NT_Q - _NT_K, 0, _NT_K - 1))),
            pl.BlockSpec((_TN, _TN), lambda m, n: (0, 0)),
            pl.BlockSpec((_S, _HD), lambda m, n: (0, 0)),
            pl.BlockSpec((_S, _HD), lambda m, n: (0, 0)),
        ],
        out_specs=pl.BlockSpec((_M // 2, _TN), lambda m, n: (m, n)),
        out_shape=jax.ShapeDtypeStruct((_M, _NTOT), jnp.bfloat16),
        compiler_params=pltpu.CompilerParams(
            dimension_semantics=(pltpu.ARBITRARY, pltpu.ARBITRARY),
        ),
    )(xb, wq, wk, wv, pswap, cos_i, sin_i)

    attn = pl.pallas_call(
        _attn_body,
        grid=(2, _H),
        in_specs=[
            pl.BlockSpec((_S, _HD), lambda b, h: (b, h)),
            pl.BlockSpec((_S, _HD), lambda b, h: (b, _H + h // _NREP)),
            pl.BlockSpec((_S, _HD), lambda b, h: (b, _H + _KVH + h // _NREP)),
        ],
        out_specs=pl.BlockSpec((_S, _HD), lambda b, h: (b, h)),
        out_shape=jax.ShapeDtypeStruct((_M, _NQ), jnp.bfloat16),
        compiler_params=pltpu.CompilerParams(
            dimension_semantics=(pltpu.PARALLEL, pltpu.ARBITRARY),
        ),
    )(qkv, qkv, qkv)

    out = pl.pallas_call(
        _proj_body,
        grid=(2, _D // _TO),
        in_specs=[
            pl.BlockSpec((_M // 2, _NQ), lambda m, j: (m, 0)),
            pl.BlockSpec((_NQ, _TO), lambda m, j: (0, j)),
        ],
        out_specs=pl.BlockSpec((_M // 2, _TO), lambda m, j: (m, j)),
        out_shape=jax.ShapeDtypeStruct((_M, _D), jnp.float32),
        compiler_params=pltpu.CompilerParams(
            dimension_semantics=(pltpu.ARBITRARY, pltpu.ARBITRARY),
        ),
    )(attn, wo)

    return out.reshape(_B, _S, _D)
